# baseline (speedup 1.0000x reference)
import functools

import jax
import jax.numpy as jnp
from jax import lax
from jax.experimental import pallas as pl
from jax.experimental.pallas import tpu as pltpu
from jax.experimental.pallas import tpu_sc as plsc

B = 4096
L = 200
LP = 208
D = 64
NC = 2
NS = 16
NW = NC * NS
RPT = B // NW
NBUF = 4
HALF = LP // 2
NLV = LP // 16


def _tile_body(x_hbm, tab_hbm, out_hbm, x_v, bufs, out_v, *sems):
    wid = lax.axis_index("s") * NC + lax.axis_index("c")
    base = wid * RPT

    pltpu.sync_copy(x_hbm.at[pl.ds(base, RPT)], x_v)

    def issue(k, r):
        pltpu.async_copy(
            tab_hbm.at[x_v.at[r, pl.ds(0, HALF)]],
            bufs.at[k, pl.ds(0, HALF)], sems[k])
        pltpu.async_copy(
            tab_hbm.at[x_v.at[r, pl.ds(HALF, HALF)]],
            bufs.at[k, pl.ds(HALF, HALF)], sems[k])

    def drain(k, r):
        pltpu.make_async_copy(
            tab_hbm.at[x_v.at[r, pl.ds(0, HALF)]],
            bufs.at[k, pl.ds(0, HALF)], sems[k]).wait()
        pltpu.make_async_copy(
            tab_hbm.at[x_v.at[r, pl.ds(HALF, HALF)]],
            bufs.at[k, pl.ds(HALF, HALF)], sems[k]).wait()

    for k in range(NBUF):
        issue(k, k)

    @pl.loop(0, RPT, step=NBUF)
    def _(g):
        for k in range(NBUF):
            r = g + k
            drain(k, r)

            cnt = jnp.zeros((16,), jnp.int32)
            for c in range(NLV):
                v = x_v[r, pl.ds(c * 16, 16)]
                cnt = cnt + plsc.all_reduce_population_count(v != 0)
            denom = jnp.maximum(cnt.astype(jnp.float32),
                                jnp.full((16,), 1.0, jnp.float32))

            zero = jnp.zeros((16,), jnp.float32)

            @plsc.parallel_loop(0, LP, unroll=8, carry=(zero,) * 4)
            def acc(l, a):
                return tuple(a[d] + bufs[k, l, pl.ds(d * 16, 16)]
                             for d in range(4))

            for d in range(4):
                out_v[r, pl.ds(d * 16, 16)] = acc[d] / denom

            @pl.when(r + NBUF < RPT)
            def _():
                issue(k, r + NBUF)

    pltpu.sync_copy(out_v, out_hbm.at[pl.ds(base, RPT)])


@jax.jit
def _run(xp, table):
    mesh = plsc.VectorSubcoreMesh(core_axis_name="c", subcore_axis_name="s")
    grid_kernel = functools.partial(
        pl.kernel,
        out_type=jax.ShapeDtypeStruct((B, D), jnp.float32),
        mesh=mesh,
        compiler_params=pltpu.CompilerParams(use_tc_tiling_on_sc=False,
                                             needs_layout_passes=False),
        scratch_types=[
            pltpu.VMEM((RPT, LP), jnp.int32),
            pltpu.VMEM((NBUF, LP, D), jnp.float32),
            pltpu.VMEM((RPT, D), jnp.float32),
            pltpu.SemaphoreType.DMA,
            pltpu.SemaphoreType.DMA,
            pltpu.SemaphoreType.DMA,
            pltpu.SemaphoreType.DMA,
        ],
    )(_tile_body)
    return grid_kernel(xp, table)


def kernel(x, table):
    xp = jnp.pad(x.astype(jnp.int32), ((0, 0), (0, LP - L)))
    return _run(xp, table)


# speedup vs baseline: 1.9619x; 1.9619x over previous
import functools

import jax
import jax.numpy as jnp
from jax import lax
from jax.experimental import pallas as pl
from jax.experimental.pallas import tpu as pltpu
from jax.experimental.pallas import tpu_sc as plsc

B = 4096
L = 200
D = 64
NC = 2
NS = 16
NW = NC * NS
RPT = B // NW
NBUF = 4
S0 = 104
S1 = L - S0
NFV = 192 // 16


def _tile_body(x_hbm, tab_hbm, out_hbm, x_v, bufs, out_v, *sems):
    wid = lax.axis_index("s") * NC + lax.axis_index("c")
    base = wid * RPT

    pltpu.sync_copy(x_hbm.at[pl.ds(base, RPT)], x_v)

    def issue(k, r):
        pltpu.async_copy(
            tab_hbm.at[x_v.at[r, pl.ds(0, S0)]],
            bufs.at[k, pl.ds(0, S0)], sems[k])
        pltpu.async_copy(
            tab_hbm.at[x_v.at[r, pl.ds(S0, S1)]],
            bufs.at[k, pl.ds(S0, S1)], sems[k])

    def drain(k, r):
        pltpu.make_async_copy(
            tab_hbm.at[x_v.at[r, pl.ds(0, S0)]],
            bufs.at[k, pl.ds(0, S0)], sems[k]).wait()
        pltpu.make_async_copy(
            tab_hbm.at[x_v.at[r, pl.ds(S0, S1)]],
            bufs.at[k, pl.ds(S0, S1)], sems[k]).wait()

    for k in range(NBUF):
        issue(k, k)

    lanes = lax.iota(jnp.int32, 16)

    @pl.loop(0, RPT, step=NBUF)
    def _(g):
        for k in range(NBUF):
            r = g + k
            drain(k, r)

            cnt = jnp.zeros((16,), jnp.int32)
            for c in range(NFV):
                v = x_v[r, pl.ds(c * 16, 16)]
                cnt = cnt + plsc.all_reduce_population_count(v != 0)
            vt = x_v[r, pl.ds(184, 16)]
            cnt = cnt + plsc.all_reduce_population_count(
                (vt != 0) & (lanes >= 8))
            denom = jnp.maximum(cnt.astype(jnp.float32),
                                jnp.full((16,), 1.0, jnp.float32))

            zero = jnp.zeros((16,), jnp.float32)

            @plsc.parallel_loop(0, L, unroll=8, carry=(zero,) * 4)
            def acc(l, a):
                return tuple(a[d] + bufs[k, l, pl.ds(d * 16, 16)]
                             for d in range(4))

            for d in range(4):
                out_v[r, pl.ds(d * 16, 16)] = acc[d] / denom

            @pl.when(r + NBUF < RPT)
            def _():
                issue(k, r + NBUF)

    pltpu.sync_copy(out_v, out_hbm.at[pl.ds(base, RPT)])


@jax.jit
def _run(x, table):
    mesh = plsc.VectorSubcoreMesh(core_axis_name="c", subcore_axis_name="s")
    grid_kernel = functools.partial(
        pl.kernel,
        out_type=jax.ShapeDtypeStruct((B, D), jnp.float32),
        mesh=mesh,
        compiler_params=pltpu.CompilerParams(use_tc_tiling_on_sc=False,
                                             needs_layout_passes=False),
        scratch_types=[
            pltpu.VMEM((RPT, L), jnp.int32),
            pltpu.VMEM((NBUF, L, D), jnp.float32),
            pltpu.VMEM((RPT, D), jnp.float32),
            pltpu.SemaphoreType.DMA,
            pltpu.SemaphoreType.DMA,
            pltpu.SemaphoreType.DMA,
            pltpu.SemaphoreType.DMA,
        ],
    )(_tile_body)
    return grid_kernel(x, table)


def kernel(x, table):
    return _run(x.astype(jnp.int32), table)


# speedup vs baseline: 1.9673x; 1.0028x over previous
import functools

import jax
import jax.numpy as jnp
from jax import lax
from jax.experimental import pallas as pl
from jax.experimental.pallas import tpu as pltpu
from jax.experimental.pallas import tpu_sc as plsc

B = 4096
L = 200
D = 64
NC = 2
NS = 16
NW = NC * NS
RPT = B // NW
NBUF = 4
S0 = 104
S1 = L - S0
NFV = 192 // 16


def _tile_body(x_hbm, tab_hbm, out_hbm, x_v, bufs, out_v, *sems):
    wid = lax.axis_index("s") * NC + lax.axis_index("c")
    base = wid * RPT

    pltpu.sync_copy(x_hbm.at[pl.ds(base, RPT)], x_v)

    def issue(k, r):
        pltpu.async_copy(
            tab_hbm.at[x_v.at[r]], bufs.at[k], sems[k])

    def drain(k, r):
        pltpu.make_async_copy(
            tab_hbm.at[x_v.at[r]], bufs.at[k], sems[k]).wait()

    for k in range(NBUF):
        issue(k, k)

    lanes = lax.iota(jnp.int32, 16)

    @pl.loop(0, RPT, step=NBUF)
    def _(g):
        for k in range(NBUF):
            r = g + k
            drain(k, r)

            cnt = jnp.zeros((16,), jnp.int32)
            for c in range(NFV):
                v = x_v[r, pl.ds(c * 16, 16)]
                cnt = cnt + plsc.all_reduce_population_count(v != 0)
            vt = x_v[r, pl.ds(184, 16)]
            cnt = cnt + plsc.all_reduce_population_count(
                (vt != 0) & (lanes >= 8))
            denom = jnp.maximum(cnt.astype(jnp.float32),
                                jnp.full((16,), 1.0, jnp.float32))

            zero = jnp.zeros((16,), jnp.float32)

            @plsc.parallel_loop(0, L, unroll=8, carry=(zero,) * 4)
            def acc(l, a):
                return tuple(a[d] + bufs[k, l, pl.ds(d * 16, 16)]
                             for d in range(4))

            for d in range(4):
                out_v[r, pl.ds(d * 16, 16)] = acc[d] / denom

            @pl.when(r + NBUF < RPT)
            def _():
                issue(k, r + NBUF)

    pltpu.sync_copy(out_v, out_hbm.at[pl.ds(base, RPT)])


@jax.jit
def _run(x, table):
    mesh = plsc.VectorSubcoreMesh(core_axis_name="c", subcore_axis_name="s")
    grid_kernel = functools.partial(
        pl.kernel,
        out_type=jax.ShapeDtypeStruct((B, D), jnp.float32),
        mesh=mesh,
        compiler_params=pltpu.CompilerParams(use_tc_tiling_on_sc=False,
                                             needs_layout_passes=False),
        scratch_types=[
            pltpu.VMEM((RPT, L), jnp.int32),
            pltpu.VMEM((NBUF, L, D), jnp.float32),
            pltpu.VMEM((RPT, D), jnp.float32),
        ] + [pltpu.SemaphoreType.DMA] * NBUF,
    )(_tile_body)
    return grid_kernel(x, table)


def kernel(x, table):
    return _run(x.astype(jnp.int32), table)
